# baseline (speedup 1.0000x reference)
import functools

import jax
import jax.numpy as jnp
from jax import lax
from jax.experimental import pallas as pl
from jax.experimental.pallas import tpu as pltpu
from jax.experimental.pallas import tpu_sc as plsc

E_BLOCK = 512
D = 128
NW = 32
GCHUNK = 400


def _sc_gather_body(v2e_hbm, u2e_hbm, row_hbm, col_hbm, uv_out, rep_out,
                    idx_v, rows2, g0, g1, o0, o1):
    wid = lax.axis_index("s") * 2 + lax.axis_index("c")
    n_per_w = row_hbm.shape[0] // NW
    nch = n_per_w // GCHUNK
    base = wid * n_per_w
    gsem = (g0, g1)
    osem = (o0, o1)
    for tab, idxh, out in ((v2e_hbm, row_hbm, uv_out),
                           (u2e_hbm, col_hbm, rep_out)):
        pltpu.sync_copy(idxh.at[pl.ds(base, n_per_w)], idx_v)
        gc = [None, None]
        oc = [None, None]
        gc[0] = pltpu.async_copy(
            tab.at[idx_v.at[pl.ds(0, GCHUNK)]], rows2.at[0], gsem[0])
        for i in range(nch):
            b = i % 2
            nb = (i + 1) % 2
            if i + 1 < nch:
                if oc[nb] is not None:
                    oc[nb].wait()
                gc[nb] = pltpu.async_copy(
                    tab.at[idx_v.at[pl.ds((i + 1) * GCHUNK, GCHUNK)]],
                    rows2.at[nb], gsem[nb])
            gc[b].wait()
            oc[b] = pltpu.async_copy(
                rows2.at[b], out.at[pl.ds(base + i * GCHUNK, GCHUNK)], osem[b])
        oc[0].wait()
        oc[1].wait()


def _sc_gather(v2e_w, u2e_w, row_idxs, col_idxs):
    n_edges = row_idxs.shape[0]
    mesh = plsc.VectorSubcoreMesh(core_axis_name="c", subcore_axis_name="s")
    f = pl.kernel(
        _sc_gather_body,
        out_type=[jax.ShapeDtypeStruct((n_edges, D), jnp.float32),
                  jax.ShapeDtypeStruct((n_edges, D), jnp.float32)],
        mesh=mesh,
        scratch_types=[
            pltpu.VMEM((n_edges // NW,), jnp.int32),
            pltpu.VMEM((2, GCHUNK, D), jnp.float32),
            pltpu.SemaphoreType.DMA,
            pltpu.SemaphoreType.DMA,
            pltpu.SemaphoreType.DMA,
            pltpu.SemaphoreType.DMA,
        ],
    )
    return f(v2e_w, u2e_w, row_idxs.astype(jnp.int32),
             col_idxs.astype(jnp.int32))


N_NODES = 10000
NPAD = 10240
ROWS_PER_TILE = 640
SEG_CHUNK = 80
Q_PER_W = 320


def _sc_scatter_body(dst3d, val, o0, o1, vbuf, xstage, ibuf, spm, sv, si):
    cid = lax.axis_index("c")
    sid = lax.axis_index("s")
    wid = sid * 2 + cid
    n_chunks = val.shape[0] // NW // SEG_CHUNK
    zero16 = jnp.zeros((16,), jnp.float32)
    for r in range(SEG_CHUNK):
        for k in range(8):
            vbuf[0, r, pl.ds(16 * k, 16)] = zero16
    for j in range(ROWS_PER_TILE // SEG_CHUNK):
        pltpu.sync_copy(
            vbuf.at[0],
            spm.at[pl.ds(ROWS_PER_TILE * sid + j * SEG_CHUNK, SEG_CHUNK)])
    plsc.subcore_barrier()
    ebase = wid * (val.shape[0] // NW)
    rbase = wid * n_chunks

    w = val.shape[1]

    def chunk_body(i, _):
        eoff = pl.multiple_of(ebase + i * SEG_CHUNK, 8)
        c1 = pltpu.async_copy(dst3d.at[rbase + i], ibuf, si)
        vdst = vbuf.at[0] if w == D else xstage.at[0]
        c2 = pltpu.async_copy(val.at[pl.ds(eoff, SEG_CHUNK)], vdst, sv)
        c1.wait()
        c2.wait()
        if w != D:
            for r in range(SEG_CHUNK):
                vbuf[0, r, pl.ds(0, 16)] = xstage[0, r, :]
        pltpu.sync_copy(vbuf.at[0], spm.at[ibuf.at[0]], add=True)
        return 0

    lax.fori_loop(0, n_chunks, chunk_body, 0)
    plsc.subcore_barrier()
    rows = pl.ds(ROWS_PER_TILE * sid, ROWS_PER_TILE)
    @pl.when(cid == 0)
    def _():
        pltpu.sync_copy(spm.at[rows], o0.at[rows])
    @pl.when(cid == 1)
    def _():
        pltpu.sync_copy(spm.at[rows], o1.at[rows])


def _sc_scatter_add(dst3d, val):
    mesh = plsc.VectorSubcoreMesh(core_axis_name="c", subcore_axis_name="s")
    f = pl.kernel(
        _sc_scatter_body,
        out_type=[jax.ShapeDtypeStruct((NPAD, D), jnp.float32),
                  jax.ShapeDtypeStruct((NPAD, D), jnp.float32)],
        mesh=mesh,
        scratch_types=[
            pltpu.VMEM((1, SEG_CHUNK, D), jnp.float32),
            pltpu.VMEM((1, SEG_CHUNK, 16), jnp.float32),
            pltpu.VMEM((1, SEG_CHUNK), jnp.int32),
            pltpu.VMEM_SHARED((NPAD, D), jnp.float32),
            pltpu.SemaphoreType.DMA,
            pltpu.SemaphoreType.DMA,
        ],
    )
    return f(dst3d, val)


def _sc_combine_body(n0, n1, s0, s1, nf, a0, a1, sb0, sb1):
    cid = lax.axis_index("c")
    sid = lax.axis_index("s")
    wid = sid * 2 + cid
    nsub = Q_PER_W // SEG_CHUNK
    for j in range(nsub):
        rows = pl.ds(wid * Q_PER_W + j * SEG_CHUNK, SEG_CHUNK)
        pltpu.sync_copy(n0.at[rows], a0)
        pltpu.sync_copy(n1.at[rows], a1)
        pltpu.sync_copy(s0.at[rows], sb0)
        pltpu.sync_copy(s1.at[rows], sb1)

        def body(r, _):
            inv = 1.0 / (sb0[r, pl.ds(0, 16)] + sb1[r, pl.ds(0, 16)] + 1e-16)
            for k in range(8):
                sl = pl.ds(16 * k, 16)
                a0[r, sl] = (a0[r, sl] + a1[r, sl]) * inv
            return 0

        lax.fori_loop(0, SEG_CHUNK, body, 0)
        pltpu.sync_copy(a0, nf.at[rows])


def _sc_combine(n0, n1, s0, s1):
    mesh = plsc.VectorSubcoreMesh(core_axis_name="c", subcore_axis_name="s")
    f = pl.kernel(
        _sc_combine_body,
        out_type=jax.ShapeDtypeStruct((NPAD, D), jnp.float32),
        mesh=mesh,
        scratch_types=[
            pltpu.VMEM((SEG_CHUNK, D), jnp.float32),
            pltpu.VMEM((SEG_CHUNK, D), jnp.float32),
            pltpu.VMEM((SEG_CHUNK, D), jnp.float32),
            pltpu.VMEM((SEG_CHUNK, D), jnp.float32),
        ],
    )
    return f(n0, n1, s0, s1)


def _sc_final_body(nf, nodes_hbm, out, idxv, a0, sem):
    cid = lax.axis_index("c")
    sid = lax.axis_index("s")
    wid = sid * 2 + cid
    base = wid * Q_PER_W
    pltpu.sync_copy(nodes_hbm.at[pl.ds(base, Q_PER_W)], idxv)
    pltpu.async_copy(nf.at[idxv], a0, sem).wait()
    pltpu.sync_copy(a0, out.at[pl.ds(base, Q_PER_W)])


def _sc_final(nf, nodes_pad):
    mesh = plsc.VectorSubcoreMesh(core_axis_name="c", subcore_axis_name="s")
    f = pl.kernel(
        _sc_final_body,
        out_type=jax.ShapeDtypeStruct((NW * Q_PER_W, D), jnp.float32),
        mesh=mesh,
        scratch_types=[
            pltpu.VMEM((Q_PER_W,), jnp.int32),
            pltpu.VMEM((Q_PER_W, D), jnp.float32),
            pltpu.SemaphoreType.DMA,
        ],
    )
    return f(nf, nodes_pad)


def _tc_dense_body(rat_ref, uv_ref, rep_ref,
                   w1a_ref, r2e8_w1b_ref, b1_ref, w2_ref, b2_ref,
                   aw1a_ref, aw1b_ref, ab1_ref, aw2_ref, ab2_ref, aw3_ref,
                   eoh_ref, exl_ref):
    f32 = jnp.float32
    bf = jnp.bfloat16
    rat = rat_ref[0, 0, :]
    oh = (rat[:, None] == lax.broadcasted_iota(jnp.int32, (E_BLOCK, 8), 1))
    oh = oh.astype(f32)
    emb_r_part = jnp.dot(oh, r2e8_w1b_ref[:], preferred_element_type=f32)
    h = jnp.dot(uv_ref[:].astype(bf), w1a_ref[:].astype(bf),
                preferred_element_type=f32)
    h = jnp.maximum(h + emb_r_part + b1_ref[0, :], 0.0)
    ohist = jnp.dot(h.astype(bf), w2_ref[:].astype(bf),
                    preferred_element_type=f32)
    ohist = jnp.maximum(ohist + b2_ref[0, :], 0.0)
    a = jnp.dot(ohist.astype(bf), aw1a_ref[:].astype(bf),
                preferred_element_type=f32)
    a = a + jnp.dot(rep_ref[:].astype(bf), aw1b_ref[:].astype(bf),
                    preferred_element_type=f32)
    a = jnp.maximum(a + ab1_ref[0, :], 0.0)
    a = jnp.dot(a.astype(bf), aw2_ref[:].astype(bf),
                preferred_element_type=f32)
    a = jnp.maximum(a + ab2_ref[0, :], 0.0)
    logits = jnp.sum(a * aw3_ref[0, :], axis=1)
    ex = jnp.exp(logits)
    eoh_ref[:] = ohist * ex[:, None]
    exl_ref[:] = jnp.broadcast_to(ex[:, None], (E_BLOCK, 16))


def _tc_dense(ratings, emb_uv, rep, mlp_w1, mlp_b1, mlp_w2, mlp_b2,
              att_w1, att_b1, att_w2, att_b2, att_w3, r2e_w):
    n_edges = emb_uv.shape[0]
    nblk = n_edges // E_BLOCK
    rat3 = ratings.astype(jnp.int32).reshape(nblk, 1, E_BLOCK)
    w1a = mlp_w1[:D]
    r2e8 = jnp.zeros((8, D), jnp.float32).at[:r2e_w.shape[0]].set(r2e_w)
    r2e8_w1b = r2e8 @ mlp_w1[D:]
    aw1a = att_w1[:D]
    aw1b = att_w1[D:]
    full = lambda shp: pl.BlockSpec(shp, lambda i: (0,) * len(shp))
    ohist, logits3 = pl.pallas_call(
        _tc_dense_body,
        grid=(nblk,),
        in_specs=[
            pl.BlockSpec((1, 1, E_BLOCK), lambda i: (i, 0, 0)),
            pl.BlockSpec((E_BLOCK, D), lambda i: (i, 0)),
            pl.BlockSpec((E_BLOCK, D), lambda i: (i, 0)),
            full((D, D)), full((8, D)), full((1, D)), full((D, D)),
            full((1, D)), full((D, D)), full((D, D)), full((1, D)),
            full((D, D)), full((1, D)), full((1, D)),
        ],
        out_specs=[
            pl.BlockSpec((E_BLOCK, D), lambda i: (i, 0)),
            pl.BlockSpec((E_BLOCK, 16), lambda i: (i, 0)),
        ],
        out_shape=[
            jax.ShapeDtypeStruct((n_edges, D), jnp.float32),
            jax.ShapeDtypeStruct((n_edges, 16), jnp.float32),
        ],
        compiler_params=pltpu.CompilerParams(
            dimension_semantics=("parallel",)),
    )(rat3, emb_uv, rep,
      w1a, r2e8_w1b, mlp_b1.reshape(1, D), mlp_w2, mlp_b2.reshape(1, D),
      aw1a, aw1b, att_b1.reshape(1, D), att_w2, att_b2.reshape(1, D),
      att_w3.reshape(1, D))
    return ohist, logits3


def kernel(nodes, edge_dst, row_idxs, col_idxs, ratings,
           v2e_w, u2e_w, r2e_w,
           mlp_w1, mlp_b1, mlp_w2, mlp_b2,
           att_w1, att_b1, att_w2, att_b2, att_w3, att_b3):
    emb_uv, rep = _sc_gather(v2e_w, u2e_w, row_idxs, col_idxs)
    eoh, exl16 = _tc_dense(ratings, emb_uv, rep,
                           mlp_w1, mlp_b1, mlp_w2, mlp_b2,
                           att_w1, att_b1, att_w2, att_b2, att_w3, r2e_w)
    n_edges = edge_dst.shape[0]
    dst3d = edge_dst.astype(jnp.int32).reshape(n_edges // SEG_CHUNK, 1, SEG_CHUNK)
    n0, n1 = _sc_scatter_add(dst3d, eoh)
    s0, s1 = _sc_scatter_add(dst3d, exl16)
    nf = _sc_combine(n0, n1, s0, s1)
    nodes_pad = jnp.concatenate(
        [nodes.astype(jnp.int32),
         jnp.zeros((NW * Q_PER_W - nodes.shape[0],), jnp.int32)])
    feat_pad = _sc_final(nf, nodes_pad)
    return feat_pad[:nodes.shape[0]]


# speedup vs baseline: 1.2834x; 1.2834x over previous
import functools

import jax
import jax.numpy as jnp
from jax import lax
from jax.experimental import pallas as pl
from jax.experimental.pallas import tpu as pltpu
from jax.experimental.pallas import tpu_sc as plsc

E_BLOCK = 2000
D = 128
NW = 32
GCHUNK = 400


def _sc_gather_body(v2e_hbm, u2e_hbm, row_hbm, col_hbm, uv_out, rep_out,
                    idx_v, rows2, g0, g1, o0, o1):
    wid = lax.axis_index("s") * 2 + lax.axis_index("c")
    n_per_w = row_hbm.shape[0] // NW
    nch = n_per_w // GCHUNK
    base = wid * n_per_w
    gsem = (g0, g1)
    osem = (o0, o1)
    for tab, idxh, out in ((v2e_hbm, row_hbm, uv_out),
                           (u2e_hbm, col_hbm, rep_out)):
        pltpu.sync_copy(idxh.at[pl.ds(base, n_per_w)], idx_v)
        gc = [None, None]
        oc = [None, None]
        gc[0] = pltpu.async_copy(
            tab.at[idx_v.at[pl.ds(0, GCHUNK)]], rows2.at[0], gsem[0])
        for i in range(nch):
            b = i % 2
            nb = (i + 1) % 2
            if i + 1 < nch:
                if oc[nb] is not None:
                    oc[nb].wait()
                gc[nb] = pltpu.async_copy(
                    tab.at[idx_v.at[pl.ds((i + 1) * GCHUNK, GCHUNK)]],
                    rows2.at[nb], gsem[nb])
            gc[b].wait()
            oc[b] = pltpu.async_copy(
                rows2.at[b], out.at[pl.ds(base + i * GCHUNK, GCHUNK)], osem[b])
        oc[0].wait()
        oc[1].wait()


def _sc_gather(v2e_w, u2e_w, row_idxs, col_idxs):
    n_edges = row_idxs.shape[0]
    mesh = plsc.VectorSubcoreMesh(core_axis_name="c", subcore_axis_name="s")
    f = pl.kernel(
        _sc_gather_body,
        out_type=[jax.ShapeDtypeStruct((n_edges, D), jnp.float32),
                  jax.ShapeDtypeStruct((n_edges, D), jnp.float32)],
        mesh=mesh,
        scratch_types=[
            pltpu.VMEM((n_edges // NW,), jnp.int32),
            pltpu.VMEM((2, GCHUNK, D), jnp.float32),
            pltpu.SemaphoreType.DMA,
            pltpu.SemaphoreType.DMA,
            pltpu.SemaphoreType.DMA,
            pltpu.SemaphoreType.DMA,
        ],
    )
    return f(v2e_w, u2e_w, row_idxs.astype(jnp.int32),
             col_idxs.astype(jnp.int32))


N_NODES = 10000
NPAD = 10240
ROWS_PER_TILE = 640
SEG_CHUNK = 80
Q_PER_W = 320


def _sc_scatter_body(dst3d, val, o0, o1, vbuf, xstage, ibuf, spm, sv, si):
    cid = lax.axis_index("c")
    sid = lax.axis_index("s")
    wid = sid * 2 + cid
    n_chunks = val.shape[0] // NW // SEG_CHUNK
    zero16 = jnp.zeros((16,), jnp.float32)
    for r in range(SEG_CHUNK):
        for k in range(8):
            vbuf[0, r, pl.ds(16 * k, 16)] = zero16
    for j in range(ROWS_PER_TILE // SEG_CHUNK):
        pltpu.sync_copy(
            vbuf.at[0],
            spm.at[pl.ds(ROWS_PER_TILE * sid + j * SEG_CHUNK, SEG_CHUNK)])
    plsc.subcore_barrier()
    ebase = wid * (val.shape[0] // NW)
    rbase = wid * n_chunks

    w = val.shape[1]

    def chunk_body(i, _):
        eoff = pl.multiple_of(ebase + i * SEG_CHUNK, 8)
        c1 = pltpu.async_copy(dst3d.at[rbase + i], ibuf, si)
        vdst = vbuf.at[0] if w == D else xstage.at[0]
        c2 = pltpu.async_copy(val.at[pl.ds(eoff, SEG_CHUNK)], vdst, sv)
        c1.wait()
        c2.wait()
        if w != D:
            for r in range(SEG_CHUNK):
                vbuf[0, r, pl.ds(0, 16)] = xstage[0, r, :]
        pltpu.sync_copy(vbuf.at[0], spm.at[ibuf.at[0]], add=True)
        return 0

    lax.fori_loop(0, n_chunks, chunk_body, 0)
    plsc.subcore_barrier()
    rows = pl.ds(ROWS_PER_TILE * sid, ROWS_PER_TILE)
    @pl.when(cid == 0)
    def _():
        pltpu.sync_copy(spm.at[rows], o0.at[rows])
    @pl.when(cid == 1)
    def _():
        pltpu.sync_copy(spm.at[rows], o1.at[rows])


def _sc_scatter_add(dst3d, val):
    mesh = plsc.VectorSubcoreMesh(core_axis_name="c", subcore_axis_name="s")
    f = pl.kernel(
        _sc_scatter_body,
        out_type=[jax.ShapeDtypeStruct((NPAD, D), jnp.float32),
                  jax.ShapeDtypeStruct((NPAD, D), jnp.float32)],
        mesh=mesh,
        scratch_types=[
            pltpu.VMEM((1, SEG_CHUNK, D), jnp.float32),
            pltpu.VMEM((1, SEG_CHUNK, 16), jnp.float32),
            pltpu.VMEM((1, SEG_CHUNK), jnp.int32),
            pltpu.VMEM_SHARED((NPAD, D), jnp.float32),
            pltpu.SemaphoreType.DMA,
            pltpu.SemaphoreType.DMA,
        ],
    )
    return f(dst3d, val)


def _sc_combine_body(n0, n1, s0, s1, nf, a0, a1, sb0, sb1):
    cid = lax.axis_index("c")
    sid = lax.axis_index("s")
    wid = sid * 2 + cid
    nsub = Q_PER_W // SEG_CHUNK
    for j in range(nsub):
        rows = pl.ds(wid * Q_PER_W + j * SEG_CHUNK, SEG_CHUNK)
        pltpu.sync_copy(n0.at[rows], a0)
        pltpu.sync_copy(n1.at[rows], a1)
        pltpu.sync_copy(s0.at[rows], sb0)
        pltpu.sync_copy(s1.at[rows], sb1)

        def body(r, _):
            inv = 1.0 / (sb0[r, pl.ds(0, 16)] + sb1[r, pl.ds(0, 16)] + 1e-16)
            for k in range(8):
                sl = pl.ds(16 * k, 16)
                a0[r, sl] = (a0[r, sl] + a1[r, sl]) * inv
            return 0

        lax.fori_loop(0, SEG_CHUNK, body, 0)
        pltpu.sync_copy(a0, nf.at[rows])


def _sc_combine(n0, n1, s0, s1):
    mesh = plsc.VectorSubcoreMesh(core_axis_name="c", subcore_axis_name="s")
    f = pl.kernel(
        _sc_combine_body,
        out_type=jax.ShapeDtypeStruct((NPAD, D), jnp.float32),
        mesh=mesh,
        scratch_types=[
            pltpu.VMEM((SEG_CHUNK, D), jnp.float32),
            pltpu.VMEM((SEG_CHUNK, D), jnp.float32),
            pltpu.VMEM((SEG_CHUNK, D), jnp.float32),
            pltpu.VMEM((SEG_CHUNK, D), jnp.float32),
        ],
    )
    return f(n0, n1, s0, s1)


def _sc_final_body(nf, nodes_hbm, out, idxv, a0, sem):
    cid = lax.axis_index("c")
    sid = lax.axis_index("s")
    wid = sid * 2 + cid
    base = wid * Q_PER_W
    pltpu.sync_copy(nodes_hbm.at[pl.ds(base, Q_PER_W)], idxv)
    pltpu.async_copy(nf.at[idxv], a0, sem).wait()
    pltpu.sync_copy(a0, out.at[pl.ds(base, Q_PER_W)])


def _sc_final(nf, nodes_pad):
    mesh = plsc.VectorSubcoreMesh(core_axis_name="c", subcore_axis_name="s")
    f = pl.kernel(
        _sc_final_body,
        out_type=jax.ShapeDtypeStruct((NW * Q_PER_W, D), jnp.float32),
        mesh=mesh,
        scratch_types=[
            pltpu.VMEM((Q_PER_W,), jnp.int32),
            pltpu.VMEM((Q_PER_W, D), jnp.float32),
            pltpu.SemaphoreType.DMA,
        ],
    )
    return f(nf, nodes_pad)


def _tc_dense_body(rat_ref, uv_ref, rep_ref,
                   w1a_ref, r2e8_w1b_ref, b1_ref, w2_ref, b2_ref,
                   aw1a_ref, aw1b_ref, ab1_ref, aw2_ref, ab2_ref, aw3_ref,
                   eoh_ref, exl_ref):
    f32 = jnp.float32
    bf = jnp.bfloat16
    rat = rat_ref[0, 0, :]
    oh = (rat[:, None] == lax.broadcasted_iota(jnp.int32, (E_BLOCK, 8), 1))
    oh = oh.astype(f32)
    emb_r_part = jnp.dot(oh, r2e8_w1b_ref[:], preferred_element_type=f32)
    h = jnp.dot(uv_ref[:].astype(bf), w1a_ref[:].astype(bf),
                preferred_element_type=f32)
    h = jnp.maximum(h + emb_r_part + b1_ref[0, :], 0.0)
    ohist = jnp.dot(h.astype(bf), w2_ref[:].astype(bf),
                    preferred_element_type=f32)
    ohist = jnp.maximum(ohist + b2_ref[0, :], 0.0)
    a = jnp.dot(ohist.astype(bf), aw1a_ref[:].astype(bf),
                preferred_element_type=f32)
    a = a + jnp.dot(rep_ref[:].astype(bf), aw1b_ref[:].astype(bf),
                    preferred_element_type=f32)
    a = jnp.maximum(a + ab1_ref[0, :], 0.0)
    a = jnp.dot(a.astype(bf), aw2_ref[:].astype(bf),
                preferred_element_type=f32)
    a = jnp.maximum(a + ab2_ref[0, :], 0.0)
    logits = jnp.sum(a * aw3_ref[0, :], axis=1)
    ex = jnp.exp(logits)
    eoh_ref[:] = ohist * ex[:, None]
    exl_ref[:] = jnp.broadcast_to(ex[:, None], (E_BLOCK, 16))


def _tc_dense(ratings, emb_uv, rep, mlp_w1, mlp_b1, mlp_w2, mlp_b2,
              att_w1, att_b1, att_w2, att_b2, att_w3, r2e_w):
    n_edges = emb_uv.shape[0]
    nblk = n_edges // E_BLOCK
    rat3 = ratings.astype(jnp.int32).reshape(nblk, 1, E_BLOCK)
    w1a = mlp_w1[:D]
    r2e8 = jnp.zeros((8, D), jnp.float32).at[:r2e_w.shape[0]].set(r2e_w)
    r2e8_w1b = r2e8 @ mlp_w1[D:]
    aw1a = att_w1[:D]
    aw1b = att_w1[D:]
    full = lambda shp: pl.BlockSpec(shp, lambda i: (0,) * len(shp))
    ohist, logits3 = pl.pallas_call(
        _tc_dense_body,
        grid=(nblk,),
        in_specs=[
            pl.BlockSpec((1, 1, E_BLOCK), lambda i: (i, 0, 0)),
            pl.BlockSpec((E_BLOCK, D), lambda i: (i, 0)),
            pl.BlockSpec((E_BLOCK, D), lambda i: (i, 0)),
            full((D, D)), full((8, D)), full((1, D)), full((D, D)),
            full((1, D)), full((D, D)), full((D, D)), full((1, D)),
            full((D, D)), full((1, D)), full((1, D)),
        ],
        out_specs=[
            pl.BlockSpec((E_BLOCK, D), lambda i: (i, 0)),
            pl.BlockSpec((E_BLOCK, 16), lambda i: (i, 0)),
        ],
        out_shape=[
            jax.ShapeDtypeStruct((n_edges, D), jnp.float32),
            jax.ShapeDtypeStruct((n_edges, 16), jnp.float32),
        ],
        compiler_params=pltpu.CompilerParams(
            dimension_semantics=("parallel",)),
    )(rat3, emb_uv, rep,
      w1a, r2e8_w1b, mlp_b1.reshape(1, D), mlp_w2, mlp_b2.reshape(1, D),
      aw1a, aw1b, att_b1.reshape(1, D), att_w2, att_b2.reshape(1, D),
      att_w3.reshape(1, D))
    return ohist, logits3


def kernel(nodes, edge_dst, row_idxs, col_idxs, ratings,
           v2e_w, u2e_w, r2e_w,
           mlp_w1, mlp_b1, mlp_w2, mlp_b2,
           att_w1, att_b1, att_w2, att_b2, att_w3, att_b3):
    emb_uv, rep = _sc_gather(v2e_w, u2e_w, row_idxs, col_idxs)
    eoh, exl16 = _tc_dense(ratings, emb_uv, rep,
                           mlp_w1, mlp_b1, mlp_w2, mlp_b2,
                           att_w1, att_b1, att_w2, att_b2, att_w3, r2e_w)
    n_edges = edge_dst.shape[0]
    dst3d = edge_dst.astype(jnp.int32).reshape(n_edges // SEG_CHUNK, 1, SEG_CHUNK)
    n0, n1 = _sc_scatter_add(dst3d, eoh)
    s0, s1 = _sc_scatter_add(dst3d, exl16)
    nf = _sc_combine(n0, n1, s0, s1)
    nodes_pad = jnp.concatenate(
        [nodes.astype(jnp.int32),
         jnp.zeros((NW * Q_PER_W - nodes.shape[0],), jnp.int32)])
    feat_pad = _sc_final(nf, nodes_pad)
    return feat_pad[:nodes.shape[0]]


# speedup vs baseline: 1.4392x; 1.1214x over previous
import functools

import jax
import jax.numpy as jnp
from jax import lax
from jax.experimental import pallas as pl
from jax.experimental.pallas import tpu as pltpu
from jax.experimental.pallas import tpu_sc as plsc

E_BLOCK = 4000
D = 128
NW = 32
GCHUNK = 400


def _sc_gather_body(v2e_hbm, u2e_hbm, row_hbm, col_hbm, uv_out, rep_out,
                    idx_v, rows2, g0, g1, o0, o1):
    wid = lax.axis_index("s") * 2 + lax.axis_index("c")
    n_per_w = row_hbm.shape[0] // NW
    nch = n_per_w // GCHUNK
    base = wid * n_per_w
    gsem = (g0, g1)
    osem = (o0, o1)
    for tab, idxh, out in ((v2e_hbm, row_hbm, uv_out),
                           (u2e_hbm, col_hbm, rep_out)):
        pltpu.sync_copy(idxh.at[pl.ds(base, n_per_w)], idx_v)
        gc = [None, None]
        oc = [None, None]
        gc[0] = pltpu.async_copy(
            tab.at[idx_v.at[pl.ds(0, GCHUNK)]], rows2.at[0], gsem[0])
        for i in range(nch):
            b = i % 2
            nb = (i + 1) % 2
            if i + 1 < nch:
                if oc[nb] is not None:
                    oc[nb].wait()
                gc[nb] = pltpu.async_copy(
                    tab.at[idx_v.at[pl.ds((i + 1) * GCHUNK, GCHUNK)]],
                    rows2.at[nb], gsem[nb])
            gc[b].wait()
            oc[b] = pltpu.async_copy(
                rows2.at[b], out.at[pl.ds(base + i * GCHUNK, GCHUNK)], osem[b])
        oc[0].wait()
        oc[1].wait()


def _sc_gather(v2e_w, u2e_w, row_idxs, col_idxs):
    n_edges = row_idxs.shape[0]
    mesh = plsc.VectorSubcoreMesh(core_axis_name="c", subcore_axis_name="s")
    f = pl.kernel(
        _sc_gather_body,
        out_type=[jax.ShapeDtypeStruct((n_edges, D), jnp.float32),
                  jax.ShapeDtypeStruct((n_edges, D), jnp.float32)],
        mesh=mesh,
        scratch_types=[
            pltpu.VMEM((n_edges // NW,), jnp.int32),
            pltpu.VMEM((2, GCHUNK, D), jnp.float32),
            pltpu.SemaphoreType.DMA,
            pltpu.SemaphoreType.DMA,
            pltpu.SemaphoreType.DMA,
            pltpu.SemaphoreType.DMA,
        ],
    )
    return f(v2e_w, u2e_w, row_idxs.astype(jnp.int32),
             col_idxs.astype(jnp.int32))


N_NODES = 10000
NPAD = 10240
ROWS_PER_TILE = 640
SEG_CHUNK = 80
Q_PER_W = 320


def _sc_scatter_body(dst3d, val, o0, o1, vbuf, xstage, ibuf, spm, sv, si):
    cid = lax.axis_index("c")
    sid = lax.axis_index("s")
    wid = sid * 2 + cid
    n_chunks = val.shape[0] // NW // SEG_CHUNK
    zero16 = jnp.zeros((16,), jnp.float32)
    for r in range(SEG_CHUNK):
        for k in range(8):
            vbuf[0, r, pl.ds(16 * k, 16)] = zero16
    for j in range(ROWS_PER_TILE // SEG_CHUNK):
        pltpu.sync_copy(
            vbuf.at[0],
            spm.at[pl.ds(ROWS_PER_TILE * sid + j * SEG_CHUNK, SEG_CHUNK)])
    plsc.subcore_barrier()
    ebase = wid * (val.shape[0] // NW)
    rbase = wid * n_chunks

    w = val.shape[1]

    def chunk_body(i, _):
        eoff = pl.multiple_of(ebase + i * SEG_CHUNK, 8)
        c1 = pltpu.async_copy(dst3d.at[rbase + i], ibuf, si)
        vdst = vbuf.at[0] if w == D else xstage.at[0]
        c2 = pltpu.async_copy(val.at[pl.ds(eoff, SEG_CHUNK)], vdst, sv)
        c1.wait()
        c2.wait()
        if w != D:
            for r in range(SEG_CHUNK):
                vbuf[0, r, pl.ds(0, 16)] = xstage[0, r, :]
        pltpu.sync_copy(vbuf.at[0], spm.at[ibuf.at[0]], add=True)
        return 0

    lax.fori_loop(0, n_chunks, chunk_body, 0)
    plsc.subcore_barrier()
    rows = pl.ds(ROWS_PER_TILE * sid, ROWS_PER_TILE)
    @pl.when(cid == 0)
    def _():
        pltpu.sync_copy(spm.at[rows], o0.at[rows])
    @pl.when(cid == 1)
    def _():
        pltpu.sync_copy(spm.at[rows], o1.at[rows])


def _sc_scatter_add(dst3d, val):
    mesh = plsc.VectorSubcoreMesh(core_axis_name="c", subcore_axis_name="s")
    f = pl.kernel(
        _sc_scatter_body,
        out_type=[jax.ShapeDtypeStruct((NPAD, D), jnp.float32),
                  jax.ShapeDtypeStruct((NPAD, D), jnp.float32)],
        mesh=mesh,
        scratch_types=[
            pltpu.VMEM((1, SEG_CHUNK, D), jnp.float32),
            pltpu.VMEM((1, SEG_CHUNK, 16), jnp.float32),
            pltpu.VMEM((1, SEG_CHUNK), jnp.int32),
            pltpu.VMEM_SHARED((NPAD, D), jnp.float32),
            pltpu.SemaphoreType.DMA,
            pltpu.SemaphoreType.DMA,
        ],
    )
    return f(dst3d, val)


def _sc_combine_body(n0, n1, s0, s1, nf, a0, a1, sb0, sb1):
    cid = lax.axis_index("c")
    sid = lax.axis_index("s")
    wid = sid * 2 + cid
    nsub = Q_PER_W // SEG_CHUNK
    for j in range(nsub):
        rows = pl.ds(wid * Q_PER_W + j * SEG_CHUNK, SEG_CHUNK)
        pltpu.sync_copy(n0.at[rows], a0)
        pltpu.sync_copy(n1.at[rows], a1)
        pltpu.sync_copy(s0.at[rows], sb0)
        pltpu.sync_copy(s1.at[rows], sb1)

        def body(r, _):
            inv = 1.0 / (sb0[r, pl.ds(0, 16)] + sb1[r, pl.ds(0, 16)] + 1e-16)
            for k in range(8):
                sl = pl.ds(16 * k, 16)
                a0[r, sl] = (a0[r, sl] + a1[r, sl]) * inv
            return 0

        lax.fori_loop(0, SEG_CHUNK, body, 0)
        pltpu.sync_copy(a0, nf.at[rows])


def _sc_combine(n0, n1, s0, s1):
    mesh = plsc.VectorSubcoreMesh(core_axis_name="c", subcore_axis_name="s")
    f = pl.kernel(
        _sc_combine_body,
        out_type=jax.ShapeDtypeStruct((NPAD, D), jnp.float32),
        mesh=mesh,
        scratch_types=[
            pltpu.VMEM((SEG_CHUNK, D), jnp.float32),
            pltpu.VMEM((SEG_CHUNK, D), jnp.float32),
            pltpu.VMEM((SEG_CHUNK, D), jnp.float32),
            pltpu.VMEM((SEG_CHUNK, D), jnp.float32),
        ],
    )
    return f(n0, n1, s0, s1)


def _sc_final_body(nf, nodes_hbm, out, idxv, a0, sem):
    cid = lax.axis_index("c")
    sid = lax.axis_index("s")
    wid = sid * 2 + cid
    base = wid * Q_PER_W
    pltpu.sync_copy(nodes_hbm.at[pl.ds(base, Q_PER_W)], idxv)
    pltpu.async_copy(nf.at[idxv], a0, sem).wait()
    pltpu.sync_copy(a0, out.at[pl.ds(base, Q_PER_W)])


def _sc_final(nf, nodes_pad):
    mesh = plsc.VectorSubcoreMesh(core_axis_name="c", subcore_axis_name="s")
    f = pl.kernel(
        _sc_final_body,
        out_type=jax.ShapeDtypeStruct((NW * Q_PER_W, D), jnp.float32),
        mesh=mesh,
        scratch_types=[
            pltpu.VMEM((Q_PER_W,), jnp.int32),
            pltpu.VMEM((Q_PER_W, D), jnp.float32),
            pltpu.SemaphoreType.DMA,
        ],
    )
    return f(nf, nodes_pad)


def _tc_dense_body(rat_ref, uv_ref, rep_ref,
                   w1a_ref, r2e8_w1b_ref, b1_ref, w2_ref, b2_ref,
                   aw1a_ref, aw1b_ref, ab1_ref, aw2_ref, ab2_ref, aw3_ref,
                   eoh_ref, exl_ref):
    f32 = jnp.float32
    bf = jnp.bfloat16
    rat = rat_ref[0, 0, :]
    oh = (rat[:, None] == lax.broadcasted_iota(jnp.int32, (E_BLOCK, 8), 1))
    oh = oh.astype(f32)
    emb_r_part = jnp.dot(oh, r2e8_w1b_ref[:], preferred_element_type=f32)
    h = jnp.dot(uv_ref[:].astype(bf), w1a_ref[:].astype(bf),
                preferred_element_type=f32)
    h = jnp.maximum(h + emb_r_part + b1_ref[0, :], 0.0)
    ohist = jnp.dot(h.astype(bf), w2_ref[:].astype(bf),
                    preferred_element_type=f32)
    ohist = jnp.maximum(ohist + b2_ref[0, :], 0.0)
    a = jnp.dot(ohist.astype(bf), aw1a_ref[:].astype(bf),
                preferred_element_type=f32)
    a = a + jnp.dot(rep_ref[:].astype(bf), aw1b_ref[:].astype(bf),
                    preferred_element_type=f32)
    a = jnp.maximum(a + ab1_ref[0, :], 0.0)
    a = jnp.dot(a.astype(bf), aw2_ref[:].astype(bf),
                preferred_element_type=f32)
    a = jnp.maximum(a + ab2_ref[0, :], 0.0)
    logits = jnp.sum(a * aw3_ref[0, :], axis=1)
    ex = jnp.exp(logits)
    eoh_ref[:] = ohist * ex[:, None]
    exl_ref[:] = jnp.broadcast_to(ex[:, None], (E_BLOCK, 16))


def _tc_dense(ratings, emb_uv, rep, mlp_w1, mlp_b1, mlp_w2, mlp_b2,
              att_w1, att_b1, att_w2, att_b2, att_w3, r2e_w):
    n_edges = emb_uv.shape[0]
    nblk = n_edges // E_BLOCK
    rat3 = ratings.astype(jnp.int32).reshape(nblk, 1, E_BLOCK)
    w1a = mlp_w1[:D]
    r2e8 = jnp.zeros((8, D), jnp.float32).at[:r2e_w.shape[0]].set(r2e_w)
    r2e8_w1b = r2e8 @ mlp_w1[D:]
    aw1a = att_w1[:D]
    aw1b = att_w1[D:]
    full = lambda shp: pl.BlockSpec(shp, lambda i: (0,) * len(shp))
    ohist, logits3 = pl.pallas_call(
        _tc_dense_body,
        grid=(nblk,),
        in_specs=[
            pl.BlockSpec((1, 1, E_BLOCK), lambda i: (i, 0, 0)),
            pl.BlockSpec((E_BLOCK, D), lambda i: (i, 0)),
            pl.BlockSpec((E_BLOCK, D), lambda i: (i, 0)),
            full((D, D)), full((8, D)), full((1, D)), full((D, D)),
            full((1, D)), full((D, D)), full((D, D)), full((1, D)),
            full((D, D)), full((1, D)), full((1, D)),
        ],
        out_specs=[
            pl.BlockSpec((E_BLOCK, D), lambda i: (i, 0)),
            pl.BlockSpec((E_BLOCK, 16), lambda i: (i, 0)),
        ],
        out_shape=[
            jax.ShapeDtypeStruct((n_edges, D), jnp.float32),
            jax.ShapeDtypeStruct((n_edges, 16), jnp.float32),
        ],
        compiler_params=pltpu.CompilerParams(
            dimension_semantics=("parallel",)),
    )(rat3, emb_uv, rep,
      w1a, r2e8_w1b, mlp_b1.reshape(1, D), mlp_w2, mlp_b2.reshape(1, D),
      aw1a, aw1b, att_b1.reshape(1, D), att_w2, att_b2.reshape(1, D),
      att_w3.reshape(1, D))
    return ohist, logits3


def kernel(nodes, edge_dst, row_idxs, col_idxs, ratings,
           v2e_w, u2e_w, r2e_w,
           mlp_w1, mlp_b1, mlp_w2, mlp_b2,
           att_w1, att_b1, att_w2, att_b2, att_w3, att_b3):
    emb_uv, rep = _sc_gather(v2e_w, u2e_w, row_idxs, col_idxs)
    eoh, exl16 = _tc_dense(ratings, emb_uv, rep,
                           mlp_w1, mlp_b1, mlp_w2, mlp_b2,
                           att_w1, att_b1, att_w2, att_b2, att_w3, r2e_w)
    n_edges = edge_dst.shape[0]
    dst3d = edge_dst.astype(jnp.int32).reshape(n_edges // SEG_CHUNK, 1, SEG_CHUNK)
    n0, n1 = _sc_scatter_add(dst3d, eoh)
    s0, s1 = _sc_scatter_add(dst3d, exl16)
    nf = _sc_combine(n0, n1, s0, s1)
    nodes_pad = jnp.concatenate(
        [nodes.astype(jnp.int32),
         jnp.zeros((NW * Q_PER_W - nodes.shape[0],), jnp.int32)])
    feat_pad = _sc_final(nf, nodes_pad)
    return feat_pad[:nodes.shape[0]]


# speedup vs baseline: 1.4744x; 1.0245x over previous
import functools

import jax
import jax.numpy as jnp
from jax import lax
from jax.experimental import pallas as pl
from jax.experimental.pallas import tpu as pltpu
from jax.experimental.pallas import tpu_sc as plsc

E_BLOCK = 8000
D = 128
NW = 32
GCHUNK = 400


def _sc_gather_body(v2e_hbm, u2e_hbm, row_hbm, col_hbm, uv_out, rep_out,
                    idx_v, rows2, g0, g1, o0, o1):
    wid = lax.axis_index("s") * 2 + lax.axis_index("c")
    n_per_w = row_hbm.shape[0] // NW
    nch = n_per_w // GCHUNK
    base = wid * n_per_w
    gsem = (g0, g1)
    osem = (o0, o1)
    for tab, idxh, out in ((v2e_hbm, row_hbm, uv_out),
                           (u2e_hbm, col_hbm, rep_out)):
        pltpu.sync_copy(idxh.at[pl.ds(base, n_per_w)], idx_v)
        gc = [None, None]
        oc = [None, None]
        gc[0] = pltpu.async_copy(
            tab.at[idx_v.at[pl.ds(0, GCHUNK)]], rows2.at[0], gsem[0])
        for i in range(nch):
            b = i % 2
            nb = (i + 1) % 2
            if i + 1 < nch:
                if oc[nb] is not None:
                    oc[nb].wait()
                gc[nb] = pltpu.async_copy(
                    tab.at[idx_v.at[pl.ds((i + 1) * GCHUNK, GCHUNK)]],
                    rows2.at[nb], gsem[nb])
            gc[b].wait()
            oc[b] = pltpu.async_copy(
                rows2.at[b], out.at[pl.ds(base + i * GCHUNK, GCHUNK)], osem[b])
        oc[0].wait()
        oc[1].wait()


def _sc_gather(v2e_w, u2e_w, row_idxs, col_idxs):
    n_edges = row_idxs.shape[0]
    mesh = plsc.VectorSubcoreMesh(core_axis_name="c", subcore_axis_name="s")
    f = pl.kernel(
        _sc_gather_body,
        out_type=[jax.ShapeDtypeStruct((n_edges, D), jnp.float32),
                  jax.ShapeDtypeStruct((n_edges, D), jnp.float32)],
        mesh=mesh,
        scratch_types=[
            pltpu.VMEM((n_edges // NW,), jnp.int32),
            pltpu.VMEM((2, GCHUNK, D), jnp.float32),
            pltpu.SemaphoreType.DMA,
            pltpu.SemaphoreType.DMA,
            pltpu.SemaphoreType.DMA,
            pltpu.SemaphoreType.DMA,
        ],
    )
    return f(v2e_w, u2e_w, row_idxs.astype(jnp.int32),
             col_idxs.astype(jnp.int32))


N_NODES = 10000
NPAD = 10240
ROWS_PER_TILE = 640
SEG_CHUNK = 80
Q_PER_W = 320


def _sc_scatter_body(dst3d, val, o0, o1, vbuf, xstage, ibuf, spm, sv, si):
    cid = lax.axis_index("c")
    sid = lax.axis_index("s")
    wid = sid * 2 + cid
    n_chunks = val.shape[0] // NW // SEG_CHUNK
    zero16 = jnp.zeros((16,), jnp.float32)
    for r in range(SEG_CHUNK):
        for k in range(8):
            vbuf[0, r, pl.ds(16 * k, 16)] = zero16
    for j in range(ROWS_PER_TILE // SEG_CHUNK):
        pltpu.sync_copy(
            vbuf.at[0],
            spm.at[pl.ds(ROWS_PER_TILE * sid + j * SEG_CHUNK, SEG_CHUNK)])
    plsc.subcore_barrier()
    ebase = wid * (val.shape[0] // NW)
    rbase = wid * n_chunks

    w = val.shape[1]

    def chunk_body(i, _):
        eoff = pl.multiple_of(ebase + i * SEG_CHUNK, 8)
        c1 = pltpu.async_copy(dst3d.at[rbase + i], ibuf, si)
        vdst = vbuf.at[0] if w == D else xstage.at[0]
        c2 = pltpu.async_copy(val.at[pl.ds(eoff, SEG_CHUNK)], vdst, sv)
        c1.wait()
        c2.wait()
        if w != D:
            for r in range(SEG_CHUNK):
                vbuf[0, r, pl.ds(0, 16)] = xstage[0, r, :]
        pltpu.sync_copy(vbuf.at[0], spm.at[ibuf.at[0]], add=True)
        return 0

    lax.fori_loop(0, n_chunks, chunk_body, 0)
    plsc.subcore_barrier()
    rows = pl.ds(ROWS_PER_TILE * sid, ROWS_PER_TILE)
    @pl.when(cid == 0)
    def _():
        pltpu.sync_copy(spm.at[rows], o0.at[rows])
    @pl.when(cid == 1)
    def _():
        pltpu.sync_copy(spm.at[rows], o1.at[rows])


def _sc_scatter_add(dst3d, val):
    mesh = plsc.VectorSubcoreMesh(core_axis_name="c", subcore_axis_name="s")
    f = pl.kernel(
        _sc_scatter_body,
        out_type=[jax.ShapeDtypeStruct((NPAD, D), jnp.float32),
                  jax.ShapeDtypeStruct((NPAD, D), jnp.float32)],
        mesh=mesh,
        scratch_types=[
            pltpu.VMEM((1, SEG_CHUNK, D), jnp.float32),
            pltpu.VMEM((1, SEG_CHUNK, 16), jnp.float32),
            pltpu.VMEM((1, SEG_CHUNK), jnp.int32),
            pltpu.VMEM_SHARED((NPAD, D), jnp.float32),
            pltpu.SemaphoreType.DMA,
            pltpu.SemaphoreType.DMA,
        ],
    )
    return f(dst3d, val)


def _sc_combine_body(n0, n1, s0, s1, nf, a0, a1, sb0, sb1):
    cid = lax.axis_index("c")
    sid = lax.axis_index("s")
    wid = sid * 2 + cid
    nsub = Q_PER_W // SEG_CHUNK
    for j in range(nsub):
        rows = pl.ds(wid * Q_PER_W + j * SEG_CHUNK, SEG_CHUNK)
        pltpu.sync_copy(n0.at[rows], a0)
        pltpu.sync_copy(n1.at[rows], a1)
        pltpu.sync_copy(s0.at[rows], sb0)
        pltpu.sync_copy(s1.at[rows], sb1)

        def body(r, _):
            inv = 1.0 / (sb0[r, pl.ds(0, 16)] + sb1[r, pl.ds(0, 16)] + 1e-16)
            for k in range(8):
                sl = pl.ds(16 * k, 16)
                a0[r, sl] = (a0[r, sl] + a1[r, sl]) * inv
            return 0

        lax.fori_loop(0, SEG_CHUNK, body, 0)
        pltpu.sync_copy(a0, nf.at[rows])


def _sc_combine(n0, n1, s0, s1):
    mesh = plsc.VectorSubcoreMesh(core_axis_name="c", subcore_axis_name="s")
    f = pl.kernel(
        _sc_combine_body,
        out_type=jax.ShapeDtypeStruct((NPAD, D), jnp.float32),
        mesh=mesh,
        scratch_types=[
            pltpu.VMEM((SEG_CHUNK, D), jnp.float32),
            pltpu.VMEM((SEG_CHUNK, D), jnp.float32),
            pltpu.VMEM((SEG_CHUNK, D), jnp.float32),
            pltpu.VMEM((SEG_CHUNK, D), jnp.float32),
        ],
    )
    return f(n0, n1, s0, s1)


def _sc_final_body(nf, nodes_hbm, out, idxv, a0, sem):
    cid = lax.axis_index("c")
    sid = lax.axis_index("s")
    wid = sid * 2 + cid
    base = wid * Q_PER_W
    pltpu.sync_copy(nodes_hbm.at[pl.ds(base, Q_PER_W)], idxv)
    pltpu.async_copy(nf.at[idxv], a0, sem).wait()
    pltpu.sync_copy(a0, out.at[pl.ds(base, Q_PER_W)])


def _sc_final(nf, nodes_pad):
    mesh = plsc.VectorSubcoreMesh(core_axis_name="c", subcore_axis_name="s")
    f = pl.kernel(
        _sc_final_body,
        out_type=jax.ShapeDtypeStruct((NW * Q_PER_W, D), jnp.float32),
        mesh=mesh,
        scratch_types=[
            pltpu.VMEM((Q_PER_W,), jnp.int32),
            pltpu.VMEM((Q_PER_W, D), jnp.float32),
            pltpu.SemaphoreType.DMA,
        ],
    )
    return f(nf, nodes_pad)


def _tc_dense_body(rat_ref, uv_ref, rep_ref,
                   w1a_ref, r2e8_w1b_ref, b1_ref, w2_ref, b2_ref,
                   aw1a_ref, aw1b_ref, ab1_ref, aw2_ref, ab2_ref, aw3_ref,
                   eoh_ref, exl_ref):
    f32 = jnp.float32
    bf = jnp.bfloat16
    rat = rat_ref[0, 0, :]
    oh = (rat[:, None] == lax.broadcasted_iota(jnp.int32, (E_BLOCK, 8), 1))
    oh = oh.astype(f32)
    emb_r_part = jnp.dot(oh, r2e8_w1b_ref[:], preferred_element_type=f32)
    h = jnp.dot(uv_ref[:].astype(bf), w1a_ref[:].astype(bf),
                preferred_element_type=f32)
    h = jnp.maximum(h + emb_r_part + b1_ref[0, :], 0.0)
    ohist = jnp.dot(h.astype(bf), w2_ref[:].astype(bf),
                    preferred_element_type=f32)
    ohist = jnp.maximum(ohist + b2_ref[0, :], 0.0)
    a = jnp.dot(ohist.astype(bf), aw1a_ref[:].astype(bf),
                preferred_element_type=f32)
    a = a + jnp.dot(rep_ref[:].astype(bf), aw1b_ref[:].astype(bf),
                    preferred_element_type=f32)
    a = jnp.maximum(a + ab1_ref[0, :], 0.0)
    a = jnp.dot(a.astype(bf), aw2_ref[:].astype(bf),
                preferred_element_type=f32)
    a = jnp.maximum(a + ab2_ref[0, :], 0.0)
    logits = jnp.sum(a * aw3_ref[0, :], axis=1)
    ex = jnp.exp(logits)
    eoh_ref[:] = ohist * ex[:, None]
    exl_ref[:] = jnp.broadcast_to(ex[:, None], (E_BLOCK, 16))


def _tc_dense(ratings, emb_uv, rep, mlp_w1, mlp_b1, mlp_w2, mlp_b2,
              att_w1, att_b1, att_w2, att_b2, att_w3, r2e_w):
    n_edges = emb_uv.shape[0]
    nblk = n_edges // E_BLOCK
    rat3 = ratings.astype(jnp.int32).reshape(nblk, 1, E_BLOCK)
    w1a = mlp_w1[:D]
    r2e8 = jnp.zeros((8, D), jnp.float32).at[:r2e_w.shape[0]].set(r2e_w)
    r2e8_w1b = r2e8 @ mlp_w1[D:]
    aw1a = att_w1[:D]
    aw1b = att_w1[D:]
    full = lambda shp: pl.BlockSpec(shp, lambda i: (0,) * len(shp))
    ohist, logits3 = pl.pallas_call(
        _tc_dense_body,
        grid=(nblk,),
        in_specs=[
            pl.BlockSpec((1, 1, E_BLOCK), lambda i: (i, 0, 0)),
            pl.BlockSpec((E_BLOCK, D), lambda i: (i, 0)),
            pl.BlockSpec((E_BLOCK, D), lambda i: (i, 0)),
            full((D, D)), full((8, D)), full((1, D)), full((D, D)),
            full((1, D)), full((D, D)), full((D, D)), full((1, D)),
            full((D, D)), full((1, D)), full((1, D)),
        ],
        out_specs=[
            pl.BlockSpec((E_BLOCK, D), lambda i: (i, 0)),
            pl.BlockSpec((E_BLOCK, 16), lambda i: (i, 0)),
        ],
        out_shape=[
            jax.ShapeDtypeStruct((n_edges, D), jnp.float32),
            jax.ShapeDtypeStruct((n_edges, 16), jnp.float32),
        ],
        compiler_params=pltpu.CompilerParams(
            dimension_semantics=("parallel",)),
    )(rat3, emb_uv, rep,
      w1a, r2e8_w1b, mlp_b1.reshape(1, D), mlp_w2, mlp_b2.reshape(1, D),
      aw1a, aw1b, att_b1.reshape(1, D), att_w2, att_b2.reshape(1, D),
      att_w3.reshape(1, D))
    return ohist, logits3


def kernel(nodes, edge_dst, row_idxs, col_idxs, ratings,
           v2e_w, u2e_w, r2e_w,
           mlp_w1, mlp_b1, mlp_w2, mlp_b2,
           att_w1, att_b1, att_w2, att_b2, att_w3, att_b3):
    emb_uv, rep = _sc_gather(v2e_w, u2e_w, row_idxs, col_idxs)
    eoh, exl16 = _tc_dense(ratings, emb_uv, rep,
                           mlp_w1, mlp_b1, mlp_w2, mlp_b2,
                           att_w1, att_b1, att_w2, att_b2, att_w3, r2e_w)
    n_edges = edge_dst.shape[0]
    dst3d = edge_dst.astype(jnp.int32).reshape(n_edges // SEG_CHUNK, 1, SEG_CHUNK)
    n0, n1 = _sc_scatter_add(dst3d, eoh)
    s0, s1 = _sc_scatter_add(dst3d, exl16)
    nf = _sc_combine(n0, n1, s0, s1)
    nodes_pad = jnp.concatenate(
        [nodes.astype(jnp.int32),
         jnp.zeros((NW * Q_PER_W - nodes.shape[0],), jnp.int32)])
    feat_pad = _sc_final(nf, nodes_pad)
    return feat_pad[:nodes.shape[0]]


# speedup vs baseline: 1.4949x; 1.0139x over previous
import functools

import jax
import jax.numpy as jnp
from jax import lax
from jax.experimental import pallas as pl
from jax.experimental.pallas import tpu as pltpu
from jax.experimental.pallas import tpu_sc as plsc

E_BLOCK = 8000
D = 128
NW = 32
GCHUNK = 400


def _sc_gather_body(v2e_hbm, u2e_hbm, row_hbm, col_hbm, uv_out, rep_out,
                    idx_v, rows2, g0, g1, o0, o1):
    wid = lax.axis_index("s") * 2 + lax.axis_index("c")
    n_per_w = row_hbm.shape[0] // NW
    nch = n_per_w // GCHUNK
    base = wid * n_per_w
    gsem = (g0, g1)
    osem = (o0, o1)
    for tab, idxh, out in ((v2e_hbm, row_hbm, uv_out),
                           (u2e_hbm, col_hbm, rep_out)):
        pltpu.sync_copy(idxh.at[pl.ds(base, n_per_w)], idx_v)
        gc = [None, None]
        oc = [None, None]
        gc[0] = pltpu.async_copy(
            tab.at[idx_v.at[pl.ds(0, GCHUNK)]], rows2.at[0], gsem[0])
        for i in range(nch):
            b = i % 2
            nb = (i + 1) % 2
            if i + 1 < nch:
                if oc[nb] is not None:
                    oc[nb].wait()
                gc[nb] = pltpu.async_copy(
                    tab.at[idx_v.at[pl.ds((i + 1) * GCHUNK, GCHUNK)]],
                    rows2.at[nb], gsem[nb])
            gc[b].wait()
            oc[b] = pltpu.async_copy(
                rows2.at[b], out.at[pl.ds(base + i * GCHUNK, GCHUNK)], osem[b])
        oc[0].wait()
        oc[1].wait()


def _sc_gather(v2e_w, u2e_w, row_idxs, col_idxs):
    n_edges = row_idxs.shape[0]
    mesh = plsc.VectorSubcoreMesh(core_axis_name="c", subcore_axis_name="s")
    f = pl.kernel(
        _sc_gather_body,
        out_type=[jax.ShapeDtypeStruct((n_edges, D), jnp.float32),
                  jax.ShapeDtypeStruct((n_edges, D), jnp.float32)],
        mesh=mesh,
        scratch_types=[
            pltpu.VMEM((n_edges // NW,), jnp.int32),
            pltpu.VMEM((2, GCHUNK, D), jnp.float32),
            pltpu.SemaphoreType.DMA,
            pltpu.SemaphoreType.DMA,
            pltpu.SemaphoreType.DMA,
            pltpu.SemaphoreType.DMA,
        ],
    )
    return f(v2e_w, u2e_w, row_idxs.astype(jnp.int32),
             col_idxs.astype(jnp.int32))


N_NODES = 10000
NPAD = 10240
ROWS_PER_TILE = 640
SEG_CHUNK = 80
Q_PER_W = 320


SC_CHUNK = 64


def _sc_scatter_body(dst3d, val, o0, o1, vbuf, xstage, ibuf, spm,
                     si0, si1, sv0, sv1, ss0, ss1):
    cid = lax.axis_index("c")
    sid = lax.axis_index("s")
    wid = sid * 2 + cid
    w = val.shape[1]
    n_chunks = val.shape[0] // SC_CHUNK
    zero16 = jnp.zeros((16,), jnp.float32)
    for r in range(SC_CHUNK):
        for k in range(8):
            vbuf[0, r, pl.ds(16 * k, 16)] = zero16
    for j in range(ROWS_PER_TILE // SC_CHUNK):
        pltpu.sync_copy(
            vbuf.at[0],
            spm.at[pl.ds(ROWS_PER_TILE * sid + j * SC_CHUNK, SC_CHUNK)])
    plsc.subcore_barrier()
    nj = (n_chunks - wid + NW - 1) // NW

    def expand(b):
        if w != D:
            for r in range(SC_CHUNK):
                vbuf[b, r, pl.ds(0, 16)] = xstage[b, r, :]

    def do_chunk(ch, b, sems):
        sia, sva, ssa = sems
        eoff = pl.multiple_of(ch * SC_CHUNK, 8)
        c1 = pltpu.async_copy(dst3d.at[ch], ibuf.at[b], sia)
        vdst = vbuf.at[b] if w == D else xstage.at[b]
        c2 = pltpu.async_copy(val.at[pl.ds(eoff, SC_CHUNK)], vdst, sva)
        c1.wait()
        c2.wait()
        expand(b)
        return pltpu.async_copy(vbuf.at[b], spm.at[ibuf.at[b, 0]], ssa,
                                add=True)

    def pair_body(jp, _):
        ch0 = wid + (2 * jp) * NW
        sc0 = do_chunk(ch0, 0, (si0, sv0, ss0))
        sc1 = do_chunk(ch0 + NW, 1, (si1, sv1, ss1))
        sc0.wait()
        sc1.wait()
        return 0

    lax.fori_loop(0, nj // 2, pair_body, 0)

    @pl.when(nj % 2 == 1)
    def _():
        do_chunk(wid + (nj - 1) * NW, 0, (si0, sv0, ss0)).wait()

    plsc.subcore_barrier()
    rows = pl.ds(ROWS_PER_TILE * sid, ROWS_PER_TILE)
    @pl.when(cid == 0)
    def _():
        pltpu.sync_copy(spm.at[rows], o0.at[rows])
    @pl.when(cid == 1)
    def _():
        pltpu.sync_copy(spm.at[rows], o1.at[rows])


def _sc_scatter_add(dst3d, val):
    mesh = plsc.VectorSubcoreMesh(core_axis_name="c", subcore_axis_name="s")
    f = pl.kernel(
        _sc_scatter_body,
        out_type=[jax.ShapeDtypeStruct((NPAD, D), jnp.float32),
                  jax.ShapeDtypeStruct((NPAD, D), jnp.float32)],
        mesh=mesh,
        scratch_types=[
            pltpu.VMEM((2, SC_CHUNK, D), jnp.float32),
            pltpu.VMEM((2, SC_CHUNK, 16), jnp.float32),
            pltpu.VMEM((2, 1, SC_CHUNK), jnp.int32),
            pltpu.VMEM_SHARED((NPAD, D), jnp.float32),
            pltpu.SemaphoreType.DMA,
            pltpu.SemaphoreType.DMA,
            pltpu.SemaphoreType.DMA,
            pltpu.SemaphoreType.DMA,
            pltpu.SemaphoreType.DMA,
            pltpu.SemaphoreType.DMA,
        ],
    )
    return f(dst3d, val)


def _sc_combine_body(n0, n1, s0, s1, nf, a0, a1, sb0, sb1):
    cid = lax.axis_index("c")
    sid = lax.axis_index("s")
    wid = sid * 2 + cid
    nsub = Q_PER_W // SEG_CHUNK
    for j in range(nsub):
        rows = pl.ds(wid * Q_PER_W + j * SEG_CHUNK, SEG_CHUNK)
        pltpu.sync_copy(n0.at[rows], a0)
        pltpu.sync_copy(n1.at[rows], a1)
        pltpu.sync_copy(s0.at[rows], sb0)
        pltpu.sync_copy(s1.at[rows], sb1)

        def body(r, _):
            inv = 1.0 / (sb0[r, pl.ds(0, 16)] + sb1[r, pl.ds(0, 16)] + 1e-16)
            for k in range(8):
                sl = pl.ds(16 * k, 16)
                a0[r, sl] = (a0[r, sl] + a1[r, sl]) * inv
            return 0

        lax.fori_loop(0, SEG_CHUNK, body, 0)
        pltpu.sync_copy(a0, nf.at[rows])


def _sc_combine(n0, n1, s0, s1):
    mesh = plsc.VectorSubcoreMesh(core_axis_name="c", subcore_axis_name="s")
    f = pl.kernel(
        _sc_combine_body,
        out_type=jax.ShapeDtypeStruct((NPAD, D), jnp.float32),
        mesh=mesh,
        scratch_types=[
            pltpu.VMEM((SEG_CHUNK, D), jnp.float32),
            pltpu.VMEM((SEG_CHUNK, D), jnp.float32),
            pltpu.VMEM((SEG_CHUNK, D), jnp.float32),
            pltpu.VMEM((SEG_CHUNK, D), jnp.float32),
        ],
    )
    return f(n0, n1, s0, s1)


def _sc_final_body(nf, nodes_hbm, out, idxv, a0, sem):
    cid = lax.axis_index("c")
    sid = lax.axis_index("s")
    wid = sid * 2 + cid
    base = wid * Q_PER_W
    pltpu.sync_copy(nodes_hbm.at[pl.ds(base, Q_PER_W)], idxv)
    pltpu.async_copy(nf.at[idxv], a0, sem).wait()
    pltpu.sync_copy(a0, out.at[pl.ds(base, Q_PER_W)])


def _sc_final(nf, nodes_pad):
    mesh = plsc.VectorSubcoreMesh(core_axis_name="c", subcore_axis_name="s")
    f = pl.kernel(
        _sc_final_body,
        out_type=jax.ShapeDtypeStruct((NW * Q_PER_W, D), jnp.float32),
        mesh=mesh,
        scratch_types=[
            pltpu.VMEM((Q_PER_W,), jnp.int32),
            pltpu.VMEM((Q_PER_W, D), jnp.float32),
            pltpu.SemaphoreType.DMA,
        ],
    )
    return f(nf, nodes_pad)


def _tc_dense_body(rat_ref, uv_ref, rep_ref,
                   w1a_ref, r2e8_w1b_ref, b1_ref, w2_ref, b2_ref,
                   aw1a_ref, aw1b_ref, ab1_ref, aw2_ref, ab2_ref, aw3_ref,
                   eoh_ref, exl_ref):
    f32 = jnp.float32
    bf = jnp.bfloat16
    rat = rat_ref[0, 0, :]
    oh = (rat[:, None] == lax.broadcasted_iota(jnp.int32, (E_BLOCK, 8), 1))
    oh = oh.astype(f32)
    emb_r_part = jnp.dot(oh, r2e8_w1b_ref[:], preferred_element_type=f32)
    h = jnp.dot(uv_ref[:].astype(bf), w1a_ref[:].astype(bf),
                preferred_element_type=f32)
    h = jnp.maximum(h + emb_r_part + b1_ref[0, :], 0.0)
    ohist = jnp.dot(h.astype(bf), w2_ref[:].astype(bf),
                    preferred_element_type=f32)
    ohist = jnp.maximum(ohist + b2_ref[0, :], 0.0)
    a = jnp.dot(ohist.astype(bf), aw1a_ref[:].astype(bf),
                preferred_element_type=f32)
    a = a + jnp.dot(rep_ref[:].astype(bf), aw1b_ref[:].astype(bf),
                    preferred_element_type=f32)
    a = jnp.maximum(a + ab1_ref[0, :], 0.0)
    a = jnp.dot(a.astype(bf), aw2_ref[:].astype(bf),
                preferred_element_type=f32)
    a = jnp.maximum(a + ab2_ref[0, :], 0.0)
    logits = jnp.sum(a * aw3_ref[0, :], axis=1)
    ex = jnp.exp(logits)
    eoh_ref[:] = ohist * ex[:, None]
    exl_ref[:] = jnp.broadcast_to(ex[:, None], (E_BLOCK, 16))


def _tc_dense(ratings, emb_uv, rep, mlp_w1, mlp_b1, mlp_w2, mlp_b2,
              att_w1, att_b1, att_w2, att_b2, att_w3, r2e_w):
    n_edges = emb_uv.shape[0]
    nblk = n_edges // E_BLOCK
    rat3 = ratings.astype(jnp.int32).reshape(nblk, 1, E_BLOCK)
    w1a = mlp_w1[:D]
    r2e8 = jnp.zeros((8, D), jnp.float32).at[:r2e_w.shape[0]].set(r2e_w)
    r2e8_w1b = r2e8 @ mlp_w1[D:]
    aw1a = att_w1[:D]
    aw1b = att_w1[D:]
    full = lambda shp: pl.BlockSpec(shp, lambda i: (0,) * len(shp))
    ohist, logits3 = pl.pallas_call(
        _tc_dense_body,
        grid=(nblk,),
        in_specs=[
            pl.BlockSpec((1, 1, E_BLOCK), lambda i: (i, 0, 0)),
            pl.BlockSpec((E_BLOCK, D), lambda i: (i, 0)),
            pl.BlockSpec((E_BLOCK, D), lambda i: (i, 0)),
            full((D, D)), full((8, D)), full((1, D)), full((D, D)),
            full((1, D)), full((D, D)), full((D, D)), full((1, D)),
            full((D, D)), full((1, D)), full((1, D)),
        ],
        out_specs=[
            pl.BlockSpec((E_BLOCK, D), lambda i: (i, 0)),
            pl.BlockSpec((E_BLOCK, 16), lambda i: (i, 0)),
        ],
        out_shape=[
            jax.ShapeDtypeStruct((n_edges, D), jnp.float32),
            jax.ShapeDtypeStruct((n_edges, 16), jnp.float32),
        ],
        compiler_params=pltpu.CompilerParams(
            dimension_semantics=("parallel",)),
    )(rat3, emb_uv, rep,
      w1a, r2e8_w1b, mlp_b1.reshape(1, D), mlp_w2, mlp_b2.reshape(1, D),
      aw1a, aw1b, att_b1.reshape(1, D), att_w2, att_b2.reshape(1, D),
      att_w3.reshape(1, D))
    return ohist, logits3


def kernel(nodes, edge_dst, row_idxs, col_idxs, ratings,
           v2e_w, u2e_w, r2e_w,
           mlp_w1, mlp_b1, mlp_w2, mlp_b2,
           att_w1, att_b1, att_w2, att_b2, att_w3, att_b3):
    emb_uv, rep = _sc_gather(v2e_w, u2e_w, row_idxs, col_idxs)
    eoh, exl16 = _tc_dense(ratings, emb_uv, rep,
                           mlp_w1, mlp_b1, mlp_w2, mlp_b2,
                           att_w1, att_b1, att_w2, att_b2, att_w3, r2e_w)
    n_edges = edge_dst.shape[0]
    dst3d = edge_dst.astype(jnp.int32).reshape(n_edges // SC_CHUNK, 1, SC_CHUNK)
    n0, n1 = _sc_scatter_add(dst3d, eoh)
    s0, s1 = _sc_scatter_add(dst3d, exl16)
    nf = _sc_combine(n0, n1, s0, s1)
    nodes_pad = jnp.concatenate(
        [nodes.astype(jnp.int32),
         jnp.zeros((NW * Q_PER_W - nodes.shape[0],), jnp.int32)])
    feat_pad = _sc_final(nf, nodes_pad)
    return feat_pad[:nodes.shape[0]]
